# Initial kernel scaffold; baseline (speedup 1.0000x reference)
import jax
import jax.numpy as jnp
from jax.experimental import pallas as pl


def kernel(x, edge_index, batch, W1, b1, W2, b2, W3, b3, W4, b4, Wout, bout):
    raise NotImplementedError("write your pallas kernel here")


# speedup vs baseline: 1.6606x; 1.6606x over previous
import jax
import jax.numpy as jnp
from jax.experimental import pallas as pl


def kernel(x, edge_index, batch, W1, b1, W2, b2, W3, b3, W4, b4, Wout, bout):
    n_nodes = x.shape[0]
    n_graphs = 64
    src = edge_index[0].astype(jnp.int32)
    dst = edge_index[1].astype(jnp.int32)
    h = x
    for W, b in ((W1, b1), (W2, b2), (W3, b3), (W4, b4)):
        F = W.shape[0] // 2
        Wd = W[:F] - W[F:]
        Wb = W[F:]
        A = h @ Wd + b
        B = h @ Wb
        smax = jax.ops.segment_max(B[src], dst, num_segments=n_nodes)
        h = jax.nn.sigmoid(A + smax)
    g = jax.ops.segment_max(h, batch, num_segments=n_graphs)
    g = jnp.where(jnp.isneginf(g), 0.0, g)
    return g @ Wout + bout


# speedup vs baseline: 3.3588x; 2.0226x over previous
import functools

import jax
import jax.numpy as jnp
from jax import lax
from jax.experimental import pallas as pl
from jax.experimental.pallas import tpu as pltpu
from jax.experimental.pallas import tpu_sc as plsc

NEG_INF = float("-inf")

CHS = 2048
FG = 2048
CH = 128
CHP = 224


_SC_PARAMS = pltpu.CompilerParams(
    needs_layout_passes=False, use_tc_tiling_on_sc=False)


def _sc_mesh():
    return plsc.VectorSubcoreMesh(core_axis_name="c", subcore_axis_name="s")


def _wid():
    return lax.axis_index("s") * lax.axis_size("c") + lax.axis_index("c")


def _make_prepro(E_pad, NPT, CAP, NW):
    n_chunks_scan = E_pad // CHS

    def body(src_hbm, dst_hbm, bsrc_hbm, bdst_hbm, nch_hbm,
             inb_s, inb_d, outb_s, outb_d, cnt_v, sem):
        wid = _wid()
        lo = wid * NPT
        hi = lo + NPT

        zero16 = jnp.zeros((16,), jnp.int32)
        npt16 = jnp.full((16,), NPT, jnp.int32)

        def init_body(i, _):
            outb_s[pl.ds(i * 16, 16)] = zero16
            outb_d[pl.ds(i * 16, 16)] = npt16
            return 0
        lax.fori_loop(0, (2 * FG) // 16, init_body, 0)

        def scan_chunk(c, carry):
            cursor, total = carry
            base = c * CHS
            pltpu.sync_copy(dst_hbm.at[pl.ds(base, CHS)], inb_d)
            pltpu.sync_copy(src_hbm.at[pl.ds(base, CHS)], inb_s)

            def scan_vec(t, cur):
                d = inb_d[pl.ds(t * 16, 16)]
                s = inb_s[pl.ds(t * 16, 16)]
                m = (d >= lo) & (d < hi)
                mi = m.astype(jnp.int32)
                excl = plsc.cumsum(mi) - mi
                idx = excl + cur
                plsc.store_scatter(outb_s, [idx], s, mask=m)
                plsc.store_scatter(outb_d, [idx], d - lo, mask=m)
                return cur + plsc.all_reduce_population_count(m)[0]

            cursor = lax.fori_loop(0, CHS // 16, scan_vec, cursor)

            do_flush = cursor >= FG

            @pl.when(do_flush)
            def _():
                off = pl.multiple_of(wid * CAP + total, FG)
                pltpu.sync_copy(outb_s.at[pl.ds(0, FG)],
                                bsrc_hbm.at[pl.ds(off, FG)])
                pltpu.sync_copy(outb_d.at[pl.ds(0, FG)],
                                bdst_hbm.at[pl.ds(off, FG)])

                def mv(i, _):
                    outb_s[pl.ds(i * 16, 16)] = outb_s[pl.ds(FG + i * 16, 16)]
                    outb_d[pl.ds(i * 16, 16)] = outb_d[pl.ds(FG + i * 16, 16)]
                    return 0
                lax.fori_loop(0, FG // 16, mv, 0)

            cursor = jnp.where(do_flush, cursor - FG, cursor)
            total = jnp.where(do_flush, total + FG, total)
            return cursor, total

        cursor, total = lax.fori_loop(
            0, n_chunks_scan, scan_chunk,
            (jnp.int32(0), jnp.int32(0)))

        has_tail = cursor > 0

        @pl.when(has_tail)
        def _():
            off = pl.multiple_of(wid * CAP + total, FG)
            pltpu.sync_copy(outb_s.at[pl.ds(0, FG)],
                            bsrc_hbm.at[pl.ds(off, FG)])
            pltpu.sync_copy(outb_d.at[pl.ds(0, FG)],
                            bdst_hbm.at[pl.ds(off, FG)])

        nflush = total // FG + has_tail.astype(jnp.int32)
        nch = nflush * (FG // CH)
        cnt_v[...] = jnp.full((16,), 1, jnp.int32) * nch
        pltpu.sync_copy(cnt_v, nch_hbm.at[pl.ds(pl.multiple_of(wid * 16, 16), 16)])

    return pl.kernel(
        body,
        out_type=(
            jax.ShapeDtypeStruct((NW * CAP,), jnp.int32),
            jax.ShapeDtypeStruct((NW * CAP,), jnp.int32),
            jax.ShapeDtypeStruct((NW * 16,), jnp.int32),
        ),
        mesh=_sc_mesh(),
        compiler_params=_SC_PARAMS,
        scratch_types=(
            pltpu.VMEM((CHS,), jnp.int32),
            pltpu.VMEM((CHS,), jnp.int32),
            pltpu.VMEM((2 * FG,), jnp.int32),
            pltpu.VMEM((2 * FG,), jnp.int32),
            pltpu.VMEM((16,), jnp.int32),
            pltpu.SemaphoreType.DMA,
        ),
        name="edge_bucket_prepro",
    )


def _make_segmax(N_pad, NPT, CAP, NW):
    def body(b_hbm, bsrc_hbm, bdst_hbm, nch_hbm, s_hbm,
             acc, srcb, dstb, rows, ncv, sem):
        wid = _wid()

        neg = jnp.full((16,), NEG_INF, jnp.float32)

        def init_body(i, _):
            for f in range(4):
                acc[i, pl.ds(16 * f, 16)] = neg
            return 0
        lax.fori_loop(0, NPT + 1, init_body, 0)

        pltpu.sync_copy(nch_hbm.at[pl.ds(pl.multiple_of(wid * 16, 16), 16)], ncv)
        nc = ncv[...][0]

        def chunk_body(c, _):
            base = pl.multiple_of(wid * CAP + c * CH, CH)
            pltpu.sync_copy(bsrc_hbm.at[pl.ds(base, CH)], srcb)
            pltpu.sync_copy(bdst_hbm.at[pl.ds(base, CH)], dstb)
            pltpu.async_copy(b_hbm.at[srcb], rows, sem).wait()

            def vec_body(t, _):
                dvec = dstb[pl.ds(t * 16, 16)]
                for j in range(16):
                    dj = dvec[j]
                    r = t * 16 + j
                    for f in range(4):
                        sl = pl.ds(16 * f, 16)
                        acc[dj, sl] = jnp.maximum(acc[dj, sl], rows[r, sl])
                return 0

            lax.fori_loop(0, CH // 16, vec_body, 0)
            return 0

        lax.fori_loop(0, nc, chunk_body, 0)
        pltpu.sync_copy(acc.at[pl.ds(0, NPT)],
                        s_hbm.at[pl.ds(pl.multiple_of(wid * NPT, 8), NPT)])

    return pl.kernel(
        body,
        out_type=jax.ShapeDtypeStruct((N_pad, 64), jnp.float32),
        mesh=_sc_mesh(),
        compiler_params=_SC_PARAMS,
        scratch_types=(
            pltpu.VMEM((NPT + 1, 64), jnp.float32),
            pltpu.VMEM((CH,), jnp.int32),
            pltpu.VMEM((CH,), jnp.int32),
            pltpu.VMEM((CH, 64), jnp.float32),
            pltpu.VMEM((16,), jnp.int32),
            pltpu.SemaphoreType.DMA,
        ),
        name="edge_segmax",
    )


def _make_pool(N_pad, NPT, NG, NW):
    NGP = 72
    def body(h_hbm, batch_hbm, part_hbm, hb, bb, part, sem):
        wid = _wid()
        base = pl.multiple_of(wid * NPT, 8)

        neg = jnp.full((16,), NEG_INF, jnp.float32)

        def init_body(g, _):
            for f in range(8):
                part[g, pl.ds(16 * f, 16)] = neg
            return 0
        lax.fori_loop(0, NGP, init_body, 0)

        pltpu.sync_copy(batch_hbm.at[pl.ds(base, NPT)], bb)

        def chunk_body(c, _):
            pltpu.sync_copy(
                h_hbm.at[pl.ds(pl.multiple_of(base + c * CHP, 8), CHP)], hb)

            def vec_body(t, _):
                gv = bb[pl.ds(c * CHP + t * 16, 16)]
                for j in range(16):
                    g = gv[j]
                    r = t * 16 + j
                    for f in range(8):
                        sl = pl.ds(16 * f, 16)
                        part[g, sl] = jnp.maximum(part[g, sl], hb[r, sl])
                return 0

            lax.fori_loop(0, CHP // 16, vec_body, 0)
            return 0

        lax.fori_loop(0, NPT // CHP, chunk_body, 0)
        pltpu.sync_copy(part, part_hbm.at[wid])

    return pl.kernel(
        body,
        out_type=jax.ShapeDtypeStruct((NW, NGP, 128), jnp.float32),
        mesh=_sc_mesh(),
        compiler_params=_SC_PARAMS,
        scratch_types=(
            pltpu.VMEM((CHP, 128), jnp.float32),
            pltpu.VMEM((NPT,), jnp.int32),
            pltpu.VMEM((NGP, 128), jnp.float32),
            pltpu.SemaphoreType.DMA,
        ),
        name="graph_max_pool",
    )


def _mm1_kernel(x_ref, wd_ref, wb_ref, b_ref, a_ref, bo_ref):
    xb = x_ref[...]
    a_ref[...] = jnp.dot(xb, wd_ref[...],
                         preferred_element_type=jnp.float32) + b_ref[...]
    bo_ref[...] = jnp.dot(xb, wb_ref[...], preferred_element_type=jnp.float32)


def _combine_mm_kernel(a_ref, s_ref, wd_ref, wb_ref, b_ref, an_ref, bn_ref):
    h = jax.nn.sigmoid(a_ref[...] + s_ref[...])
    an_ref[...] = jnp.dot(h, wd_ref[...],
                          preferred_element_type=jnp.float32) + b_ref[...]
    bn_ref[...] = jnp.dot(h, wb_ref[...], preferred_element_type=jnp.float32)


def _combine_mm4_kernel(a_ref, s_ref, wd_ref, wb_ref, b_ref,
                        an_ref, b4a_ref, b4b_ref):
    h = jax.nn.sigmoid(a_ref[...] + s_ref[...])
    an_ref[...] = jnp.dot(h, wd_ref[...],
                          preferred_element_type=jnp.float32) + b_ref[...]
    b4 = jnp.dot(h, wb_ref[...], preferred_element_type=jnp.float32)
    b4a_ref[...] = b4[:, :64]
    b4b_ref[...] = b4[:, 64:]


def _combine4_kernel(a_ref, sa_ref, sb_ref, h_ref):
    s = jnp.concatenate([sa_ref[...], sb_ref[...]], axis=1)
    h_ref[...] = jax.nn.sigmoid(a_ref[...] + s)


def _make_final(NW, NG):
    def body(p_ref, wo_ref, bo_ref, o_ref):
        p = p_ref[...]
        red = p[0]
        for i in range(1, NW):
            red = jnp.maximum(red, p[i])
        g = red[:NG]
        g = jnp.where(jnp.isneginf(g), 0.0, g)
        o_ref[...] = jnp.dot(g, wo_ref[...],
                             preferred_element_type=jnp.float32) + bo_ref[...]

    return body


def _row_block(bm, cols):
    return pl.BlockSpec((bm, cols), lambda i: (i, 0))


def _full_block(shape):
    return pl.BlockSpec(shape, lambda i: tuple(0 for _ in shape))


def kernel(x, edge_index, batch, W1, b1, W2, b2, W3, b3, W4, b4, Wout, bout):
    N = x.shape[0]
    E = edge_index.shape[1]
    NG = 64
    NW = 32

    NPT = (((N + NW - 1) // NW) + 7) // 8 * 8
    N_pad = NW * NPT
    E_pad = ((E + CHS - 1) // CHS) * CHS
    CAP = E_pad + FG

    src = edge_index[0].astype(jnp.int32)
    dst = edge_index[1].astype(jnp.int32)
    src = jnp.concatenate([src, jnp.zeros((E_pad - E,), jnp.int32)])
    dst = jnp.concatenate([dst, jnp.full((E_pad - E,), 1 << 28, jnp.int32)])
    batch_p = jnp.concatenate(
        [batch.astype(jnp.int32), jnp.full((N_pad - N,), NG, jnp.int32)])

    x8 = jnp.zeros((N_pad, 8), jnp.float32).at[:N, :3].set(x)

    prepro = _make_prepro(E_pad, NPT, CAP, NW)
    bsrc, bdst, nch = prepro(src, dst)

    segmax = _make_segmax(N_pad, NPT, CAP, NW)

    BM = 1792
    grid = (N_pad // BM,)

    def split_w(W):
        F = W.shape[0] // 2
        return W[:F] - W[F:], W[F:]

    Wd1, Wb1 = split_w(W1)
    Wd1 = jnp.zeros((8, 64), jnp.float32).at[:3].set(Wd1)
    Wb1 = jnp.zeros((8, 64), jnp.float32).at[:3].set(Wb1)

    A, B = pl.pallas_call(
        _mm1_kernel,
        grid=grid,
        in_specs=[_row_block(BM, 8), _full_block((8, 64)),
                  _full_block((8, 64)), _full_block((1, 64))],
        out_specs=[_row_block(BM, 64), _row_block(BM, 64)],
        out_shape=[jax.ShapeDtypeStruct((N_pad, 64), jnp.float32)] * 2,
    )(x8, Wd1, Wb1, b1.reshape(1, 64))

    for Wl, bl in ((W2, b2), (W3, b3)):
        S = segmax(B, bsrc, bdst, nch)
        Wd, Wb = split_w(Wl)
        A, B = pl.pallas_call(
            _combine_mm_kernel,
            grid=grid,
            in_specs=[_row_block(BM, 64), _row_block(BM, 64),
                      _full_block((64, 64)), _full_block((64, 64)),
                      _full_block((1, 64))],
            out_specs=[_row_block(BM, 64), _row_block(BM, 64)],
            out_shape=[jax.ShapeDtypeStruct((N_pad, 64), jnp.float32)] * 2,
        )(A, S, Wd, Wb, bl.reshape(1, 64))

    S = segmax(B, bsrc, bdst, nch)
    Wd4, Wb4 = split_w(W4)
    A4, B4a, B4b = pl.pallas_call(
        _combine_mm4_kernel,
        grid=grid,
        in_specs=[_row_block(BM, 64), _row_block(BM, 64),
                  _full_block((64, 128)), _full_block((64, 128)),
                  _full_block((1, 128))],
        out_specs=[_row_block(BM, 128), _row_block(BM, 64),
                   _row_block(BM, 64)],
        out_shape=[jax.ShapeDtypeStruct((N_pad, 128), jnp.float32),
                   jax.ShapeDtypeStruct((N_pad, 64), jnp.float32),
                   jax.ShapeDtypeStruct((N_pad, 64), jnp.float32)],
    )(A, S, Wd4, Wb4, b4.reshape(1, 128))

    S4a = segmax(B4a, bsrc, bdst, nch)
    S4b = segmax(B4b, bsrc, bdst, nch)

    h4 = pl.pallas_call(
        _combine4_kernel,
        grid=grid,
        in_specs=[_row_block(BM, 128), _row_block(BM, 64),
                  _row_block(BM, 64)],
        out_specs=_row_block(BM, 128),
        out_shape=jax.ShapeDtypeStruct((N_pad, 128), jnp.float32),
    )(A4, S4a, S4b)

    pool = _make_pool(N_pad, NPT, NG, NW)
    partials = pool(h4, batch_p)

    out = pl.pallas_call(
        _make_final(NW, NG),
        out_shape=jax.ShapeDtypeStruct((NG, 10), jnp.float32),
    )(partials, Wout, bout.reshape(1, 10))

    return out
